# baseline (speedup 1.0000x reference)
import jax
import jax.numpy as jnp
from jax import lax
from jax.experimental import pallas as pl
from jax.experimental.pallas import tpu as pltpu
from jax.experimental.pallas import tpu_sc as plsc

_N = 10000
_E = 320000
_D = 128
_H = 256

_NC = 2
_NS = 16
_NW = _NC * _NS

_K = 128
_CHUNKS = 80
_PASSES = 5
_PCH = _CHUNKS // _PASSES
_EROWS = _E // _D
_EPAD = _NW * _CHUNKS
_NP = 10240
_RPT = _NP // _NS


def _build_y_body(x_ref, emb_ref, src_ref, attr_ref, y_ref, idx_ref):
    x = x_ref[...]
    for a in range(4):
        y_ref[a] = jnp.maximum(x + emb_ref[a, :][None, :], 0.0)
    idx_ref[...] = attr_ref[...] * _N + src_ref[...]


def _build_y(x, emb, src_p, attr_p):
    return pl.pallas_call(
        _build_y_body,
        out_shape=(
            jax.ShapeDtypeStruct((4, _N, _D), jnp.float32),
            jax.ShapeDtypeStruct((_EPAD, _D), jnp.int32),
        ),
    )(x, emb, src_p, attr_p)


def _sc_body(y_hbm, idx_hbm, dst_hbm, out_hbm,
             idx_v, dst_v, rows_a, rows_b, acc_sh, sem_a, sem_b, sem_sa, sem_sb):
    cid = lax.axis_index("c")
    sid = lax.axis_index("s")
    blk = cid * _NS + sid

    pltpu.async_copy(idx_hbm.at[blk, 0], idx_v, sem_a)
    pltpu.async_copy(dst_hbm.at[blk, 0], dst_v, sem_b)

    zvec = jnp.zeros((16,), jnp.float32)

    def zrow(r, carry):
        for j in range(_D // 16):
            rows_a[r, pl.ds(j * 16, 16)] = zvec
        return carry

    lax.fori_loop(0, _K, zrow, 0)
    base = sid * _RPT
    for i in range(_RPT // _K):
        pltpu.sync_copy(rows_a, acc_sh.at[pl.ds(base + i * _K, _K), :])
    rem = _RPT - (_RPT // _K) * _K
    if rem:
        pltpu.sync_copy(rows_a.at[pl.ds(0, rem), :],
                        acc_sh.at[pl.ds(base + _RPT - rem, rem), :])
    pltpu.make_async_copy(idx_hbm.at[blk, 0], idx_v, sem_a).wait()
    pltpu.make_async_copy(dst_hbm.at[blk, 0], dst_v, sem_b).wait()
    plsc.subcore_barrier()

    def gather(c, buf, sem):
        pltpu.async_copy(y_hbm.at[idx_v.at[c]], buf, sem)

    def wait_gather(c, buf, sem):
        pltpu.make_async_copy(y_hbm.at[idx_v.at[c]], buf, sem).wait()

    def scat(c, buf, sem):
        pltpu.async_copy(buf, acc_sh.at[dst_v.at[c]], sem, add=True)

    def wait_scat(c, buf, sem):
        pltpu.make_async_copy(buf, acc_sh.at[dst_v.at[c]], sem).wait()

    for p in range(_PASSES):
        if p:
            pltpu.sync_copy(idx_hbm.at[blk, p], idx_v)
            pltpu.sync_copy(dst_hbm.at[blk, p], dst_v)

        gather(0, rows_a, sem_a)

        def step(i, carry):
            c = 2 * i
            gather(c + 1, rows_b, sem_b)
            wait_gather(c, rows_a, sem_a)
            scat(c, rows_a, sem_sa)
            wait_scat(c, rows_a, sem_sa)
            gather(c + 2, rows_a, sem_a)
            wait_gather(c + 1, rows_b, sem_b)
            scat(c + 1, rows_b, sem_sb)
            wait_scat(c + 1, rows_b, sem_sb)
            return carry

        lax.fori_loop(0, _PCH // 2 - 1, step, 0)
        c = _PCH - 2
        gather(c + 1, rows_b, sem_b)
        wait_gather(c, rows_a, sem_a)
        scat(c, rows_a, sem_sa)
        wait_scat(c, rows_a, sem_sa)
        wait_gather(c + 1, rows_b, sem_b)
        scat(c + 1, rows_b, sem_sb)
        wait_scat(c + 1, rows_b, sem_sb)
    plsc.subcore_barrier()

    pltpu.sync_copy(acc_sh.at[pl.ds(base, _RPT), :],
                    out_hbm.at[cid, pl.ds(base, _RPT), :])


def _sc_agg(y_flat, idx_r, dst_r):
    kern = pl.kernel(
        _sc_body,
        out_type=jax.ShapeDtypeStruct((_NC, _NP, _D), jnp.float32),
        mesh=plsc.VectorSubcoreMesh(core_axis_name="c", subcore_axis_name="s"),
        scratch_types=[
            pltpu.VMEM((_PCH, _K), jnp.int32),
            pltpu.VMEM((_PCH, _K), jnp.int32),
            pltpu.VMEM((_K, _D), jnp.float32),
            pltpu.VMEM((_K, _D), jnp.float32),
            pltpu.VMEM_SHARED((_NP, _D), jnp.float32),
            pltpu.SemaphoreType.DMA,
            pltpu.SemaphoreType.DMA,
            pltpu.SemaphoreType.DMA,
            pltpu.SemaphoreType.DMA,
        ],
    )
    return kern(y_flat, idx_r, dst_r)


def _mlp_body(x_ref, agg_ref, eps_ref, w1_ref, g1_ref, b1_ref,
              w2_ref, g2_ref, b2_ref, w3_ref, b3_ref, y_ref):
    x = x_ref[...]
    h = (1.0 + eps_ref[0, 0]) * x + agg_ref[0, :_N] + agg_ref[1, :_N]

    h1 = jnp.dot(h, w1_ref[...], preferred_element_type=jnp.float32)
    m1 = jnp.mean(h1, axis=0)
    v1 = jnp.mean(jnp.square(h1 - m1[None, :]), axis=0)
    h1 = (h1 - m1[None, :]) * lax.rsqrt(v1 + 1e-5)[None, :]
    h1 = jnp.maximum(h1 * g1_ref[...][None, :] + b1_ref[...][None, :], 0.0)

    h2 = jnp.dot(h1, w2_ref[...], preferred_element_type=jnp.float32)
    m2 = jnp.mean(h2, axis=0)
    v2 = jnp.mean(jnp.square(h2 - m2[None, :]), axis=0)
    h2 = (h2 - m2[None, :]) * lax.rsqrt(v2 + 1e-5)[None, :]
    h2 = jnp.maximum(h2 * g2_ref[...][None, :] + b2_ref[...][None, :], 0.0)

    y = jnp.dot(h2, w3_ref[...], preferred_element_type=jnp.float32)
    y_ref[...] = y + b3_ref[...][None, :] + x


def _mlp(x, agg, eps, W1, g1, b1, W2, g2, b2, W3, b3):
    return pl.pallas_call(
        _mlp_body,
        out_shape=jax.ShapeDtypeStruct((_N, _D), jnp.float32),
    )(x, agg, eps.reshape(1, 1), W1, g1, b1, W2, g2, b2, W3, b3)


def kernel(x_P0, edge_index, edge_attr, emb, eps, W1, g1, b1, W2, g2, b2, W3, b3):
    npad = _EPAD - _EROWS
    src_p = jnp.pad(edge_index[0].reshape(_EROWS, _D), ((0, npad), (0, 0)))
    attr_p = jnp.pad(edge_attr.reshape(_EROWS, _D), ((0, npad), (0, 0)))
    pad_dst = _N + (jnp.arange(npad * _D, dtype=jnp.int32)
                    .reshape(npad, _D) % (_NP - _N))
    dst_p = jnp.concatenate(
        [edge_index[1].reshape(_EROWS, _D), pad_dst], axis=0)
    y4, idx = _build_y(x_P0, emb, src_p, attr_p)
    y_flat = y4.reshape(4 * _N, _D)
    idx_r = idx.reshape(_NW, _PASSES, _PCH, _K)
    dstp_r = dst_p.reshape(_NW, _PASSES, _PCH, _K)
    agg = _sc_agg(y_flat, idx_r, dstp_r)
    return _mlp(x_P0, agg, eps.astype(jnp.float32), W1, g1, b1, W2, g2, b2, W3, b3)


# speedup vs baseline: 2.7194x; 2.7194x over previous
import jax
import jax.numpy as jnp
from jax import lax
from jax.experimental import pallas as pl
from jax.experimental.pallas import tpu as pltpu
from jax.experimental.pallas import tpu_sc as plsc

_N = 10000
_E = 320000
_D = 128
_H = 256

_NC = 2
_NS = 16
_NW = _NC * _NS

_K = 128
_CHUNKS = 80
_PASSES = 5
_PCH = _CHUNKS // _PASSES
_EROWS = _E // _D
_EPAD = _NW * _CHUNKS
_NP = 10240
_RPT = _NP // _NS


def _build_y_body(x_ref, emb_ref, src_ref, attr_ref, y_ref, idx_ref):
    x = x_ref[...]
    for a in range(4):
        y_ref[a] = jnp.maximum(x + emb_ref[a, :][None, :], 0.0)
    idx_ref[...] = attr_ref[...] * _N + src_ref[...]


def _build_y(x, emb, src_p, attr_p):
    return pl.pallas_call(
        _build_y_body,
        out_shape=(
            jax.ShapeDtypeStruct((4, _N, _D), jnp.float32),
            jax.ShapeDtypeStruct((_EPAD, _D), jnp.int32),
        ),
    )(x, emb, src_p, attr_p)


def _sc_body(y_hbm, idx_hbm, dst_hbm, out_hbm,
             idx_v, dst_v, rows_a, rows_b, acc_sh, sem_a, sem_b, sem_sa, sem_sb):
    cid = lax.axis_index("c")
    sid = lax.axis_index("s")
    blk = cid * _NS + sid

    pltpu.async_copy(idx_hbm.at[blk, 0], idx_v, sem_a)
    pltpu.async_copy(dst_hbm.at[blk, 0], dst_v, sem_b)

    zvec = jnp.zeros((16,), jnp.float32)

    def zrow(r, carry):
        for j in range(_D // 16):
            rows_a[r, pl.ds(j * 16, 16)] = zvec
        return carry

    lax.fori_loop(0, _K, zrow, 0)
    base = sid * _RPT
    for i in range(_RPT // _K):
        pltpu.sync_copy(rows_a, acc_sh.at[pl.ds(base + i * _K, _K), :])
    rem = _RPT - (_RPT // _K) * _K
    if rem:
        pltpu.sync_copy(rows_a.at[pl.ds(0, rem), :],
                        acc_sh.at[pl.ds(base + _RPT - rem, rem), :])
    pltpu.make_async_copy(idx_hbm.at[blk, 0], idx_v, sem_a).wait()
    pltpu.make_async_copy(dst_hbm.at[blk, 0], dst_v, sem_b).wait()
    plsc.subcore_barrier()

    def gather(c, buf, sem):
        pltpu.async_copy(y_hbm.at[idx_v.at[c]], buf, sem)

    def wait_gather(c, buf, sem):
        pltpu.make_async_copy(y_hbm.at[idx_v.at[c]], buf, sem).wait()

    def scat(c, buf, sem):
        pltpu.async_copy(buf, acc_sh.at[dst_v.at[c]], sem, add=True)

    def wait_scat(c, buf, sem):
        pltpu.make_async_copy(buf, acc_sh.at[dst_v.at[c]], sem).wait()

    for p in range(_PASSES):
        if p:
            pltpu.sync_copy(idx_hbm.at[blk, p], idx_v)
            pltpu.sync_copy(dst_hbm.at[blk, p], dst_v)

        gather(0, rows_a, sem_a)

        def step(i, carry):
            c = 2 * i
            gather(c + 1, rows_b, sem_b)
            wait_gather(c, rows_a, sem_a)
            scat(c, rows_a, sem_sa)
            wait_scat(c, rows_a, sem_sa)
            gather(c + 2, rows_a, sem_a)
            wait_gather(c + 1, rows_b, sem_b)
            scat(c + 1, rows_b, sem_sb)
            wait_scat(c + 1, rows_b, sem_sb)
            return carry

        lax.fori_loop(0, _PCH // 2 - 1, step, 0)
        c = _PCH - 2
        gather(c + 1, rows_b, sem_b)
        wait_gather(c, rows_a, sem_a)
        scat(c, rows_a, sem_sa)
        wait_scat(c, rows_a, sem_sa)
        wait_gather(c + 1, rows_b, sem_b)
        scat(c + 1, rows_b, sem_sb)
        wait_scat(c + 1, rows_b, sem_sb)
    plsc.subcore_barrier()

    pltpu.sync_copy(acc_sh.at[pl.ds(base, _RPT), :],
                    out_hbm.at[cid, pl.ds(base, _RPT), :])


def _sc_agg(y_flat, idx_r, dst_r):
    kern = pl.kernel(
        _sc_body,
        out_type=jax.ShapeDtypeStruct((_NC, _NP, _D), jnp.float32),
        mesh=plsc.VectorSubcoreMesh(core_axis_name="c", subcore_axis_name="s"),
        scratch_types=[
            pltpu.VMEM((_PCH, _K), jnp.int32),
            pltpu.VMEM((_PCH, _K), jnp.int32),
            pltpu.VMEM((_K, _D), jnp.float32),
            pltpu.VMEM((_K, _D), jnp.float32),
            pltpu.VMEM_SHARED((_NP, _D), jnp.float32),
            pltpu.SemaphoreType.DMA,
            pltpu.SemaphoreType.DMA,
            pltpu.SemaphoreType.DMA,
            pltpu.SemaphoreType.DMA,
        ],
    )
    return kern(y_flat, idx_r, dst_r)


def _mlp_body(x_ref, agg_ref, eps_ref, w1_ref, g1_ref, b1_ref,
              w2_ref, g2_ref, b2_ref, w3_ref, b3_ref, y_ref):
    x = x_ref[...]
    h = (1.0 + eps_ref[0, 0]) * x + agg_ref[0, :_N] + agg_ref[1, :_N]

    h1 = jnp.dot(h, w1_ref[...], preferred_element_type=jnp.float32)
    m1 = jnp.mean(h1, axis=0)
    v1 = jnp.mean(jnp.square(h1 - m1[None, :]), axis=0)
    h1 = (h1 - m1[None, :]) * lax.rsqrt(v1 + 1e-5)[None, :]
    h1 = jnp.maximum(h1 * g1_ref[...][None, :] + b1_ref[...][None, :], 0.0)

    h2 = jnp.dot(h1, w2_ref[...], preferred_element_type=jnp.float32)
    m2 = jnp.mean(h2, axis=0)
    v2 = jnp.mean(jnp.square(h2 - m2[None, :]), axis=0)
    h2 = (h2 - m2[None, :]) * lax.rsqrt(v2 + 1e-5)[None, :]
    h2 = jnp.maximum(h2 * g2_ref[...][None, :] + b2_ref[...][None, :], 0.0)

    y = jnp.dot(h2, w3_ref[...], preferred_element_type=jnp.float32)
    y_ref[...] = y + b3_ref[...][None, :] + x


def _mlp(x, agg, eps, W1, g1, b1, W2, g2, b2, W3, b3):
    return pl.pallas_call(
        _mlp_body,
        out_shape=jax.ShapeDtypeStruct((_N, _D), jnp.float32),
    )(x, agg, eps.reshape(1, 1), W1, g1, b1, W2, g2, b2, W3, b3)


def kernel(x_P0, edge_index, edge_attr, emb, eps, W1, g1, b1, W2, g2, b2, W3, b3):
    npad = _EPAD - _EROWS
    ar = jnp.arange(npad * _D, dtype=jnp.int32).reshape(npad, _D)
    src_p = jnp.concatenate(
        [edge_index[0].reshape(_EROWS, _D), ar % _N], axis=0)
    attr_p = jnp.concatenate(
        [edge_attr.reshape(_EROWS, _D), ar % 4], axis=0)
    dst_p = jnp.concatenate(
        [edge_index[1].reshape(_EROWS, _D), _N + ar % (_NP - _N)], axis=0)
    y4, idx = _build_y(x_P0, emb, src_p, attr_p)
    y_flat = y4.reshape(4 * _N, _D)
    idx_r = idx.reshape(_NW, _PASSES, _PCH, _K)
    dstp_r = dst_p.reshape(_NW, _PASSES, _PCH, _K)
    agg = _sc_agg(y_flat, idx_r, dstp_r)
    return _mlp(x_P0, agg, eps.astype(jnp.float32), W1, g1, b1, W2, g2, b2, W3, b3)


# speedup vs baseline: 2.7994x; 1.0294x over previous
import jax
import jax.numpy as jnp
from jax import lax
from jax.experimental import pallas as pl
from jax.experimental.pallas import tpu as pltpu
from jax.experimental.pallas import tpu_sc as plsc

_N = 10000
_E = 320000
_D = 128
_H = 256

_NC = 2
_NS = 16
_NW = _NC * _NS

_K = 128
_CHUNKS = 80
_PASSES = 5
_PCH = _CHUNKS // _PASSES
_EROWS = _E // _D
_EPAD = _NW * _CHUNKS
_NP = 10240
_RPT = _NP // _NS


def _build_y_body(x_ref, emb_ref, src_ref, attr_ref, y_ref, idx_ref):
    x = x_ref[...]
    for a in range(4):
        y_ref[a] = jnp.maximum(x + emb_ref[a, :][None, :], 0.0)
    idx_ref[...] = attr_ref[...] * _N + src_ref[...]


def _build_y(x, emb, src_p, attr_p):
    return pl.pallas_call(
        _build_y_body,
        out_shape=(
            jax.ShapeDtypeStruct((4, _N, _D), jnp.float32),
            jax.ShapeDtypeStruct((_EPAD, _D), jnp.int32),
        ),
    )(x, emb, src_p, attr_p)


def _sc_body(y_hbm, idx_hbm, dst_hbm, out_hbm,
             iv0, dv0, iv1, dv1, rows_a, rows_b, acc_sh,
             sem_a, sem_b, sem_sa, sem_sb, sem_st):
    cid = lax.axis_index("c")
    sid = lax.axis_index("s")
    blk = cid * _NS + sid

    pltpu.async_copy(idx_hbm.at[blk, 0], iv0, sem_st)
    pltpu.async_copy(dst_hbm.at[blk, 0], dv0, sem_st)

    zvec = jnp.zeros((16,), jnp.float32)

    def zrow(r, carry):
        for j in range(_D // 16):
            rows_a[r, pl.ds(j * 16, 16)] = zvec
        return carry

    lax.fori_loop(0, _K, zrow, 0)
    base = sid * _RPT
    for i in range(_RPT // _K):
        pltpu.sync_copy(rows_a, acc_sh.at[pl.ds(base + i * _K, _K), :])
    rem = _RPT - (_RPT // _K) * _K
    if rem:
        pltpu.sync_copy(rows_a.at[pl.ds(0, rem), :],
                        acc_sh.at[pl.ds(base + _RPT - rem, rem), :])
    pltpu.make_async_copy(idx_hbm.at[blk, 0], iv0, sem_st).wait()
    pltpu.make_async_copy(dst_hbm.at[blk, 0], dv0, sem_st).wait()
    plsc.subcore_barrier()

    def pipeline(idx_v, dst_v):
        def gather(c, buf, sem):
            pltpu.async_copy(y_hbm.at[idx_v.at[c]], buf, sem)

        def wait_gather(c, buf, sem):
            pltpu.make_async_copy(y_hbm.at[idx_v.at[c]], buf, sem).wait()

        def scat(c, buf, sem):
            pltpu.async_copy(buf, acc_sh.at[dst_v.at[c]], sem, add=True)

        def wait_scat(c, buf, sem):
            pltpu.make_async_copy(buf, acc_sh.at[dst_v.at[c]], sem).wait()

        gather(0, rows_a, sem_a)

        def step(i, carry):
            c = 2 * i
            gather(c + 1, rows_b, sem_b)
            wait_gather(c, rows_a, sem_a)
            scat(c, rows_a, sem_sa)
            wait_scat(c, rows_a, sem_sa)
            gather(c + 2, rows_a, sem_a)
            wait_gather(c + 1, rows_b, sem_b)
            scat(c + 1, rows_b, sem_sb)
            wait_scat(c + 1, rows_b, sem_sb)
            return carry

        lax.fori_loop(0, _PCH // 2 - 1, step, 0)
        c = _PCH - 2
        gather(c + 1, rows_b, sem_b)
        wait_gather(c, rows_a, sem_a)
        scat(c, rows_a, sem_sa)
        wait_scat(c, rows_a, sem_sa)
        wait_gather(c + 1, rows_b, sem_b)
        scat(c + 1, rows_b, sem_sb)
        wait_scat(c + 1, rows_b, sem_sb)

    bufs = ((iv0, dv0), (iv1, dv1))
    for p in range(_PASSES):
        iv, dv = bufs[p % 2]
        if p + 1 < _PASSES:
            nv, ndv = bufs[(p + 1) % 2]
            pltpu.async_copy(idx_hbm.at[blk, p + 1], nv, sem_st)
            pltpu.async_copy(dst_hbm.at[blk, p + 1], ndv, sem_st)
        pipeline(iv, dv)
        if p + 1 < _PASSES:
            pltpu.make_async_copy(idx_hbm.at[blk, p + 1], nv, sem_st).wait()
            pltpu.make_async_copy(dst_hbm.at[blk, p + 1], ndv, sem_st).wait()
    plsc.subcore_barrier()

    pltpu.sync_copy(acc_sh.at[pl.ds(base, _RPT), :],
                    out_hbm.at[cid, pl.ds(base, _RPT), :])


def _sc_agg(y_flat, idx_r, dst_r):
    kern = pl.kernel(
        _sc_body,
        out_type=jax.ShapeDtypeStruct((_NC, _NP, _D), jnp.float32),
        mesh=plsc.VectorSubcoreMesh(core_axis_name="c", subcore_axis_name="s"),
        scratch_types=[
            pltpu.VMEM((_PCH, _K), jnp.int32),
            pltpu.VMEM((_PCH, _K), jnp.int32),
            pltpu.VMEM((_PCH, _K), jnp.int32),
            pltpu.VMEM((_PCH, _K), jnp.int32),
            pltpu.VMEM((_K, _D), jnp.float32),
            pltpu.VMEM((_K, _D), jnp.float32),
            pltpu.VMEM_SHARED((_NP, _D), jnp.float32),
            pltpu.SemaphoreType.DMA,
            pltpu.SemaphoreType.DMA,
            pltpu.SemaphoreType.DMA,
            pltpu.SemaphoreType.DMA,
            pltpu.SemaphoreType.DMA,
        ],
    )
    return kern(y_flat, idx_r, dst_r)


def _mlp_body(x_ref, agg_ref, eps_ref, w1_ref, g1_ref, b1_ref,
              w2_ref, g2_ref, b2_ref, w3_ref, b3_ref, y_ref):
    x = x_ref[...]
    h = (1.0 + eps_ref[0, 0]) * x + agg_ref[0, :_N] + agg_ref[1, :_N]

    h1 = jnp.dot(h.astype(jnp.bfloat16), w1_ref[...].astype(jnp.bfloat16),
                 preferred_element_type=jnp.float32)
    m1 = jnp.mean(h1, axis=0)
    v1 = jnp.mean(jnp.square(h1 - m1[None, :]), axis=0)
    h1 = (h1 - m1[None, :]) * lax.rsqrt(v1 + 1e-5)[None, :]
    h1 = jnp.maximum(h1 * g1_ref[...][None, :] + b1_ref[...][None, :], 0.0)

    h2 = jnp.dot(h1.astype(jnp.bfloat16), w2_ref[...].astype(jnp.bfloat16),
                 preferred_element_type=jnp.float32)
    m2 = jnp.mean(h2, axis=0)
    v2 = jnp.mean(jnp.square(h2 - m2[None, :]), axis=0)
    h2 = (h2 - m2[None, :]) * lax.rsqrt(v2 + 1e-5)[None, :]
    h2 = jnp.maximum(h2 * g2_ref[...][None, :] + b2_ref[...][None, :], 0.0)

    y = jnp.dot(h2.astype(jnp.bfloat16), w3_ref[...].astype(jnp.bfloat16),
                 preferred_element_type=jnp.float32)
    y_ref[...] = y + b3_ref[...][None, :] + x


def _mlp(x, agg, eps, W1, g1, b1, W2, g2, b2, W3, b3):
    return pl.pallas_call(
        _mlp_body,
        out_shape=jax.ShapeDtypeStruct((_N, _D), jnp.float32),
    )(x, agg, eps.reshape(1, 1), W1, g1, b1, W2, g2, b2, W3, b3)


def kernel(x_P0, edge_index, edge_attr, emb, eps, W1, g1, b1, W2, g2, b2, W3, b3):
    npad = _EPAD - _EROWS
    ar = jnp.arange(npad * _D, dtype=jnp.int32).reshape(npad, _D)
    src_p = jnp.concatenate(
        [edge_index[0].reshape(_EROWS, _D), ar % _N], axis=0)
    attr_p = jnp.concatenate(
        [edge_attr.reshape(_EROWS, _D), ar % 4], axis=0)
    dst_p = jnp.concatenate(
        [edge_index[1].reshape(_EROWS, _D), _N + ar % (_NP - _N)], axis=0)
    y4, idx = _build_y(x_P0, emb, src_p, attr_p)
    y_flat = y4.reshape(4 * _N, _D)
    idx_r = idx.reshape(_NW, _PASSES, _PCH, _K)
    dstp_r = dst_p.reshape(_NW, _PASSES, _PCH, _K)
    agg = _sc_agg(y_flat, idx_r, dstp_r)
    return _mlp(x_P0, agg, eps.astype(jnp.float32), W1, g1, b1, W2, g2, b2, W3, b3)
